# Initial kernel scaffold; baseline (speedup 1.0000x reference)
import jax
import jax.numpy as jnp
from jax.experimental import pallas as pl


def kernel(frames):
    raise NotImplementedError("write your pallas kernel here")


# speedup vs baseline: 2.2307x; 2.2307x over previous
import numpy as np
import jax
import jax.numpy as jnp
from jax.experimental import pallas as pl

_ALPHA = 4


def _copy_block(src_ref, dst_ref):
    dst_ref[...] = src_ref[...]


def kernel(frames):
    C, T, H, W = frames.shape
    n = T // _ALPHA
    assert tuple(np.linspace(0.0, T - 1, n).astype(np.int32)) == tuple(
        t * (T - 1) // (n - 1) for t in range(n)
    )

    slow = pl.pallas_call(
        _copy_block,
        grid=(n,),
        in_specs=[
            pl.BlockSpec((C, 1, H, W), lambda t: (0, t * (T - 1) // (n - 1), 0, 0)),
        ],
        out_specs=pl.BlockSpec((C, 1, H, W), lambda t: (0, t, 0, 0)),
        out_shape=jax.ShapeDtypeStruct((C, n, H, W), frames.dtype),
    )(frames)

    return (slow, frames)
